# baseline (speedup 1.0000x reference)
import functools

import jax
import jax.numpy as jnp
from jax import lax
from jax.experimental import pallas as pl
from jax.experimental.pallas import tpu as pltpu
from jax.experimental.pallas import tpu_sc as plsc

BATCH = 1048576
DIM = 8
NINC = 1000
ROWP = 1008
BLK = 128
NBLK = BATCH // BLK

NC = 2
NS = 16
NW = NC * NS
L = 16

SPW = BATCH // NW
CH = 2048
NCHUNK = SPW // CH
GROUPS = CH // L

_JAC_SCALE = float(NINC) ** DIM


_mesh = plsc.VectorSubcoreMesh(core_axis_name="c", subcore_axis_name="s")


@functools.partial(
    pl.kernel,
    mesh=_mesh,
    compiler_params=pltpu.CompilerParams(needs_layout_passes=False),
    out_type=[
        jax.ShapeDtypeStruct((BATCH * DIM,), jnp.float32),
        jax.ShapeDtypeStruct((BATCH,), jnp.float32),
    ],
    scratch_types=[
        pltpu.VMEM((DIM * ROWP,), jnp.int32),
        pltpu.VMEM((CH * DIM,), jnp.float32),
        pltpu.VMEM((CH * DIM,), jnp.float32),
        pltpu.VMEM((CH * DIM,), jnp.float32),
        pltpu.VMEM((CH * DIM,), jnp.float32),
        pltpu.VMEM((CH,), jnp.float32),
        pltpu.VMEM((CH,), jnp.float32),
        pltpu.SemaphoreType.DMA,
        pltpu.SemaphoreType.DMA,
        pltpu.SemaphoreType.DMA,
        pltpu.SemaphoreType.DMA,
        pltpu.SemaphoreType.DMA,
        pltpu.SemaphoreType.DMA,
    ],
)
def _vegas_sc(y_hbm, tab_hbm, x_hbm, jac_hbm,
              tab_v, y0_v, y1_v, x0_v, x1_v, j0_v, j1_v,
              sy0, sy1, sx0, sx1, sj0, sj1):
    wid = lax.axis_index("s") * NC + lax.axis_index("c")
    base = wid * SPW

    pltpu.sync_copy(tab_hbm, tab_v)

    def start_y(c, y_v, sem):
        s0 = base + c * CH
        pltpu.async_copy(y_hbm.at[pl.ds(s0 * DIM, CH * DIM)], y_v, sem)

    def wait_y(y_v, sem):
        pltpu.make_async_copy(y_hbm.at[pl.ds(0, CH * DIM)], y_v, sem).wait()

    def start_out(c, x_v, jac_v, sem_x, sem_j):
        s0 = base + c * CH
        pltpu.async_copy(x_v, x_hbm.at[pl.ds(s0 * DIM, CH * DIM)], sem_x)
        pltpu.async_copy(jac_v, jac_hbm.at[pl.ds(s0, CH)], sem_j)

    def wait_out(x_v, jac_v, sem_x, sem_j):
        pltpu.make_async_copy(x_v, x_hbm.at[pl.ds(0, CH * DIM)], sem_x).wait()
        pltpu.make_async_copy(jac_v, jac_hbm.at[pl.ds(0, CH)], sem_j).wait()

    def compute_chunk(y_v, x_v, jac_v):
        def one_group(blk, sub):
            boff = blk * (BLK * DIM) + sub * L
            ys = [y_v[pl.ds(boff + d * BLK, L)] for d in range(DIM)]
            y1000s = [ys[d] * float(NINC) for d in range(DIM)]
            iycs = [y1000s[d].astype(jnp.int32) for d in range(DIM)]
            tws = [plsc.load_gather(tab_v.at[pl.ds(d * ROWP, ROWP)],
                                    [iycs[d]]) for d in range(DIM)]
            gvs = [lax.bitcast_convert_type(tws[d], jnp.float32)
                   for d in range(DIM)]
            ivs = [lax.bitcast_convert_type(tws[d] << 16,
                                            jnp.float32) for d in range(DIM)]
            xvs = [gvs[d] + ivs[d] * y1000s[d] for d in range(DIM)]
            for d in range(DIM):
                x_v[pl.ds(boff + d * BLK, L)] = xvs[d]
            p01 = ivs[0] * ivs[1]
            p23 = ivs[2] * ivs[3]
            p45 = ivs[4] * ivs[5]
            p67 = ivs[6] * ivs[7]
            jac = (p01 * p23) * (p45 * p67)
            jac_v[pl.ds(blk * BLK + sub * L, L)] = jac * _JAC_SCALE

        @plsc.parallel_loop(0, GROUPS, unroll=2)
        def _group_body(g):
            one_group(g >> 3, g & 7)

    NPAIR = NCHUNK // 2
    start_y(0, y0_v, sy0)

    def pair_body(p, carry):
        c0 = 2 * p
        wait_y(y0_v, sy0)
        start_y(c0 + 1, y1_v, sy1)

        @pl.when(p > 0)
        def _():
            wait_out(x0_v, j0_v, sx0, sj0)

        compute_chunk(y0_v, x0_v, j0_v)
        start_out(c0, x0_v, j0_v, sx0, sj0)

        wait_y(y1_v, sy1)

        @pl.when(p < NPAIR - 1)
        def _():
            start_y(c0 + 2, y0_v, sy0)

        @pl.when(p > 0)
        def _():
            wait_out(x1_v, j1_v, sx1, sj1)

        compute_chunk(y1_v, x1_v, j1_v)
        start_out(c0 + 1, x1_v, j1_v, sx1, sj1)
        return carry

    lax.fori_loop(0, NPAIR, pair_body, 0)
    wait_out(x0_v, j0_v, sx0, sj0)
    wait_out(x1_v, j1_v, sx1, sj1)


def kernel(y, grid, inc):
    yl = y.T.reshape(DIM, NBLK, BLK).transpose(1, 0, 2).reshape(-1)
    gridp = jnp.pad(grid, ((0, 0), (0, ROWP - NINC - 1)), mode="edge")
    incp = jnp.pad(inc, ((0, 0), (0, ROWP - NINC)), mode="edge")
    ii = jnp.arange(ROWP, dtype=jnp.float32)[None, :]
    a_val = gridp - ii * incp
    b_bits = lax.bitcast_convert_type(
        incp.astype(jnp.bfloat16), jnp.uint16).astype(jnp.uint32)
    t_bits = lax.bitcast_convert_type(a_val, jnp.uint32)
    h0 = (t_bits >> 16).astype(jnp.int32)
    cands = [jnp.clip(h0 + o, 0, 0xFFFF).astype(jnp.uint32)
             for o in (-1, 0, 1)]
    cvals = [lax.bitcast_convert_type((h << 16) | b_bits, jnp.float32)
             for h in cands]
    errs = jnp.stack([jnp.abs(v - a_val) for v in cvals])
    pick = jnp.argmin(errs, axis=0)
    word = jnp.stack([(h << 16) | b_bits for h in cands])
    tab = lax.bitcast_convert_type(
        jnp.take_along_axis(word, pick[None], axis=0)[0], jnp.int32).reshape(-1)
    xl, jac = _vegas_sc(yl, tab)
    x = xl.reshape(NBLK, DIM, BLK).transpose(1, 0, 2).reshape(DIM, BATCH).T
    return x, jac


# speedup vs baseline: 1.1195x; 1.1195x over previous
import functools

import jax
import jax.numpy as jnp
from jax import lax
from jax.experimental import pallas as pl
from jax.experimental.pallas import tpu as pltpu
from jax.experimental.pallas import tpu_sc as plsc

BATCH = 1048576
DIM = 8
NINC = 1000
ROWP = 1008
BLK = 128
NBLK = BATCH // BLK

NC = 2
NS = 16
NW = NC * NS
L = 16

SPW = BATCH // NW
CH = 2048
NCHUNK = SPW // CH
GROUPS = CH // L

_JAC_SCALE = float(NINC) ** DIM


_mesh = plsc.VectorSubcoreMesh(core_axis_name="c", subcore_axis_name="s")


@functools.partial(
    pl.kernel,
    mesh=_mesh,
    compiler_params=pltpu.CompilerParams(needs_layout_passes=False),
    out_type=[
        jax.ShapeDtypeStruct((BATCH * DIM,), jnp.float32),
        jax.ShapeDtypeStruct((BATCH,), jnp.float32),
    ],
    scratch_types=[
        pltpu.VMEM((DIM * ROWP,), jnp.int32),
        pltpu.VMEM((CH * DIM,), jnp.float32),
        pltpu.VMEM((CH * DIM,), jnp.float32),
        pltpu.VMEM((CH * DIM,), jnp.float32),
        pltpu.VMEM((CH * DIM,), jnp.float32),
        pltpu.VMEM((CH,), jnp.float32),
        pltpu.VMEM((CH,), jnp.float32),
        pltpu.SemaphoreType.DMA,
        pltpu.SemaphoreType.DMA,
        pltpu.SemaphoreType.DMA,
        pltpu.SemaphoreType.DMA,
        pltpu.SemaphoreType.DMA,
        pltpu.SemaphoreType.DMA,
    ],
)
def _vegas_sc(y_hbm, tab_hbm, x_hbm, jac_hbm,
              tab_v, y0_v, y1_v, x0_v, x1_v, j0_v, j1_v,
              sy0, sy1, sx0, sx1, sj0, sj1):
    wid = lax.axis_index("s") * NC + lax.axis_index("c")
    base = wid * SPW

    pltpu.sync_copy(tab_hbm, tab_v)

    def start_y(c, y_v, sem):
        s0 = base + c * CH
        pltpu.async_copy(y_hbm.at[pl.ds(s0 * DIM, CH * DIM)], y_v, sem)

    def wait_y(y_v, sem):
        pltpu.make_async_copy(y_hbm.at[pl.ds(0, CH * DIM)], y_v, sem).wait()

    def start_out(c, x_v, jac_v, sem_x, sem_j):
        s0 = base + c * CH
        pltpu.async_copy(x_v, x_hbm.at[pl.ds(s0 * DIM, CH * DIM)], sem_x)
        pltpu.async_copy(jac_v, jac_hbm.at[pl.ds(s0, CH)], sem_j)

    def wait_out(x_v, jac_v, sem_x, sem_j):
        pltpu.make_async_copy(x_v, x_hbm.at[pl.ds(0, CH * DIM)], sem_x).wait()
        pltpu.make_async_copy(jac_v, jac_hbm.at[pl.ds(0, CH)], sem_j).wait()

    def compute_chunk(y_v, x_v, jac_v):
        def one_group(blk, sub):
            boff = blk * (BLK * DIM) + sub * L
            ys = [y_v[pl.ds(boff + d * BLK, L)] for d in range(DIM)]
            y1000s = [ys[d] * float(NINC) for d in range(DIM)]
            iycs = [y1000s[d].astype(jnp.int32) for d in range(DIM)]
            tws = [plsc.load_gather(tab_v.at[pl.ds(d * ROWP, ROWP)],
                                    [iycs[d]]) for d in range(DIM)]
            gvs = [lax.bitcast_convert_type(tws[d], jnp.float32)
                   for d in range(DIM)]
            ivs = [lax.bitcast_convert_type(tws[d] << 16,
                                            jnp.float32) for d in range(DIM)]
            xvs = [gvs[d] + ivs[d] * y1000s[d] for d in range(DIM)]
            for d in range(DIM):
                x_v[pl.ds(boff + d * BLK, L)] = xvs[d]
            p01 = ivs[0] * ivs[1]
            p23 = ivs[2] * ivs[3]
            p45 = ivs[4] * ivs[5]
            p67 = ivs[6] * ivs[7]
            jac = (p01 * p23) * (p45 * p67)
            jac_v[pl.ds(blk * BLK + sub * L, L)] = jac * _JAC_SCALE

        @plsc.parallel_loop(0, GROUPS, unroll=2)
        def _group_body(g):
            one_group(g >> 3, g & 7)

    NPAIR = NCHUNK // 2
    start_y(0, y0_v, sy0)

    def pair_body(p, carry):
        c0 = 2 * p
        wait_y(y0_v, sy0)
        start_y(c0 + 1, y1_v, sy1)

        @pl.when(p > 0)
        def _():
            wait_out(x0_v, j0_v, sx0, sj0)

        compute_chunk(y0_v, x0_v, j0_v)
        start_out(c0, x0_v, j0_v, sx0, sj0)

        wait_y(y1_v, sy1)

        @pl.when(p < NPAIR - 1)
        def _():
            start_y(c0 + 2, y0_v, sy0)

        @pl.when(p > 0)
        def _():
            wait_out(x1_v, j1_v, sx1, sj1)

        compute_chunk(y1_v, x1_v, j1_v)
        start_out(c0 + 1, x1_v, j1_v, sx1, sj1)
        return carry

    lax.fori_loop(0, NPAIR, pair_body, 0)
    wait_out(x0_v, j0_v, sx0, sj0)
    wait_out(x1_v, j1_v, sx1, sj1)


def kernel(y, grid, inc):
    yl = y.T.reshape(DIM, NBLK, BLK).transpose(1, 0, 2).reshape(-1)
    gridp = jnp.pad(grid, ((0, 0), (0, ROWP - NINC - 1)), mode="edge")
    incp = jnp.pad(inc, ((0, 0), (0, ROWP - NINC)), mode="edge")
    ii = jnp.arange(ROWP, dtype=jnp.float32)[None, :]
    a_val = gridp - ii * incp
    b_bits = lax.bitcast_convert_type(
        incp.astype(jnp.bfloat16), jnp.uint16).astype(jnp.uint32)
    t_bits = lax.bitcast_convert_type(a_val, jnp.uint32)
    h0 = (t_bits >> 16).astype(jnp.int32)
    cands = [jnp.clip(h0 + o, 0, 0xFFFF).astype(jnp.uint32)
             for o in (-1, 0, 1)]
    cvals = [lax.bitcast_convert_type((h << 16) | b_bits, jnp.float32)
             for h in cands]
    errs = [jnp.abs(v - a_val) for v in cvals]
    words = [(h << 16) | b_bits for h in cands]
    w01 = jnp.where(errs[0] < errs[1], words[0], words[1])
    e01 = jnp.minimum(errs[0], errs[1])
    best = jnp.where(errs[2] < e01, words[2], w01)
    tab = lax.bitcast_convert_type(best, jnp.int32).reshape(-1)
    xl, jac = _vegas_sc(yl, tab)
    x = xl.reshape(NBLK, DIM, BLK).transpose(1, 0, 2).reshape(DIM, BATCH).T
    return x, jac


# speedup vs baseline: 1.1296x; 1.0090x over previous
import functools

import jax
import jax.numpy as jnp
from jax import lax
from jax.experimental import pallas as pl
from jax.experimental.pallas import tpu as pltpu
from jax.experimental.pallas import tpu_sc as plsc

BATCH = 1048576
DIM = 8
NINC = 1000
ROWP = 1008
BLK = 128
NBLK = BATCH // BLK

NC = 2
NS = 16
NW = NC * NS
L = 16

SPW = BATCH // NW
CH = 2048
NCHUNK = SPW // CH
GROUPS = CH // L

_JAC_SCALE = float(NINC) ** DIM


_mesh = plsc.VectorSubcoreMesh(core_axis_name="c", subcore_axis_name="s")


@functools.partial(
    pl.kernel,
    mesh=_mesh,
    compiler_params=pltpu.CompilerParams(needs_layout_passes=False),
    out_type=[
        jax.ShapeDtypeStruct((BATCH * DIM,), jnp.float32),
        jax.ShapeDtypeStruct((BATCH,), jnp.float32),
    ],
    scratch_types=[
        pltpu.VMEM((DIM * ROWP,), jnp.int32),
        pltpu.VMEM((CH * DIM,), jnp.float32),
        pltpu.VMEM((CH * DIM,), jnp.float32),
        pltpu.VMEM((CH * DIM,), jnp.float32),
        pltpu.VMEM((CH * DIM,), jnp.float32),
        pltpu.VMEM((CH,), jnp.float32),
        pltpu.VMEM((CH,), jnp.float32),
        pltpu.SemaphoreType.DMA,
        pltpu.SemaphoreType.DMA,
        pltpu.SemaphoreType.DMA,
        pltpu.SemaphoreType.DMA,
        pltpu.SemaphoreType.DMA,
        pltpu.SemaphoreType.DMA,
    ],
)
def _vegas_sc(y_hbm, tab_hbm, x_hbm, jac_hbm,
              tab_v, y0_v, y1_v, x0_v, x1_v, j0_v, j1_v,
              sy0, sy1, sx0, sx1, sj0, sj1):
    wid = lax.axis_index("s") * NC + lax.axis_index("c")
    base = wid * SPW

    pltpu.sync_copy(tab_hbm, tab_v)

    def start_y(c, y_v, sem):
        s0 = base + c * CH
        pltpu.async_copy(y_hbm.at[pl.ds(s0 * DIM, CH * DIM)], y_v, sem)

    def wait_y(y_v, sem):
        pltpu.make_async_copy(y_hbm.at[pl.ds(0, CH * DIM)], y_v, sem).wait()

    def start_out(c, x_v, jac_v, sem_x, sem_j):
        s0 = base + c * CH
        pltpu.async_copy(x_v, x_hbm.at[pl.ds(s0 * DIM, CH * DIM)], sem_x)
        pltpu.async_copy(jac_v, jac_hbm.at[pl.ds(s0, CH)], sem_j)

    def wait_out(x_v, jac_v, sem_x, sem_j):
        pltpu.make_async_copy(x_v, x_hbm.at[pl.ds(0, CH * DIM)], sem_x).wait()
        pltpu.make_async_copy(jac_v, jac_hbm.at[pl.ds(0, CH)], sem_j).wait()

    def compute_chunk(y_v, x_v, jac_v):
        def one_group(blk, sub):
            boff = blk * (BLK * DIM) + sub * L
            ys = [y_v[pl.ds(boff + d * BLK, L)] for d in range(DIM)]
            y1000s = [ys[d] * float(NINC) for d in range(DIM)]
            iycs = [y1000s[d].astype(jnp.int32) for d in range(DIM)]
            tws = [plsc.load_gather(tab_v.at[pl.ds(d * ROWP, ROWP)],
                                    [iycs[d]]) for d in range(DIM)]
            gvs = [lax.bitcast_convert_type(tws[d], jnp.float32)
                   for d in range(DIM)]
            ivs = [lax.bitcast_convert_type(tws[d] << 16,
                                            jnp.float32) for d in range(DIM)]
            xvs = [gvs[d] + ivs[d] * ys[d] for d in range(DIM)]
            for d in range(DIM):
                x_v[pl.ds(boff + d * BLK, L)] = xvs[d]
            p01 = ivs[0] * ivs[1]
            p23 = ivs[2] * ivs[3]
            p45 = ivs[4] * ivs[5]
            p67 = ivs[6] * ivs[7]
            jac = (p01 * p23) * (p45 * p67)
            jac_v[pl.ds(blk * BLK + sub * L, L)] = jac

        @plsc.parallel_loop(0, GROUPS, unroll=2)
        def _group_body(g):
            one_group(g >> 3, g & 7)

    NPAIR = NCHUNK // 2
    start_y(0, y0_v, sy0)

    def pair_body(p, carry):
        c0 = 2 * p
        wait_y(y0_v, sy0)
        start_y(c0 + 1, y1_v, sy1)

        @pl.when(p > 0)
        def _():
            wait_out(x0_v, j0_v, sx0, sj0)

        compute_chunk(y0_v, x0_v, j0_v)
        start_out(c0, x0_v, j0_v, sx0, sj0)

        wait_y(y1_v, sy1)

        @pl.when(p < NPAIR - 1)
        def _():
            start_y(c0 + 2, y0_v, sy0)

        @pl.when(p > 0)
        def _():
            wait_out(x1_v, j1_v, sx1, sj1)

        compute_chunk(y1_v, x1_v, j1_v)
        start_out(c0 + 1, x1_v, j1_v, sx1, sj1)
        return carry

    lax.fori_loop(0, NPAIR, pair_body, 0)
    wait_out(x0_v, j0_v, sx0, sj0)
    wait_out(x1_v, j1_v, sx1, sj1)


def kernel(y, grid, inc):
    yl = y.T.reshape(DIM, NBLK, BLK).transpose(1, 0, 2).reshape(-1)
    gridp = jnp.pad(grid, ((0, 0), (0, ROWP - NINC - 1)), mode="edge")
    incp = jnp.pad(inc, ((0, 0), (0, ROWP - NINC)), mode="edge")
    ii = jnp.arange(ROWP, dtype=jnp.float32)[None, :]
    a_val = gridp - ii * incp
    b_bits = lax.bitcast_convert_type(
        (incp * float(NINC)).astype(jnp.bfloat16), jnp.uint16).astype(jnp.uint32)
    t_bits = lax.bitcast_convert_type(a_val, jnp.uint32)
    h0 = (t_bits >> 16).astype(jnp.int32)
    cands = [jnp.clip(h0 + o, 0, 0xFFFF).astype(jnp.uint32)
             for o in (-1, 0, 1)]
    cvals = [lax.bitcast_convert_type((h << 16) | b_bits, jnp.float32)
             for h in cands]
    errs = [jnp.abs(v - a_val) for v in cvals]
    words = [(h << 16) | b_bits for h in cands]
    w01 = jnp.where(errs[0] < errs[1], words[0], words[1])
    e01 = jnp.minimum(errs[0], errs[1])
    best = jnp.where(errs[2] < e01, words[2], w01)
    tab = lax.bitcast_convert_type(best, jnp.int32).reshape(-1)
    xl, jac = _vegas_sc(yl, tab)
    x = xl.reshape(NBLK, DIM, BLK).transpose(1, 0, 2).reshape(DIM, BATCH).T
    return x, jac


# speedup vs baseline: 1.1311x; 1.0013x over previous
import functools

import jax
import jax.numpy as jnp
from jax import lax
from jax.experimental import pallas as pl
from jax.experimental.pallas import tpu as pltpu
from jax.experimental.pallas import tpu_sc as plsc

BATCH = 1048576
DIM = 8
NINC = 1000
ROWP = 1008
BLK = 128
NBLK = BATCH // BLK

NC = 2
NS = 16
NW = NC * NS
L = 16

SPW = BATCH // NW
CH = 2048
NCHUNK = SPW // CH
GROUPS = CH // L

_mesh = plsc.VectorSubcoreMesh(core_axis_name="c", subcore_axis_name="s")


@functools.partial(
    pl.kernel,
    mesh=_mesh,
    compiler_params=pltpu.CompilerParams(needs_layout_passes=False),
    out_type=[
        jax.ShapeDtypeStruct((BATCH * DIM,), jnp.float32),
        jax.ShapeDtypeStruct((BATCH,), jnp.float32),
    ],
    scratch_types=[
        pltpu.VMEM((DIM * ROWP,), jnp.int32),
        pltpu.VMEM((CH * DIM,), jnp.float32),
        pltpu.VMEM((CH * DIM,), jnp.float32),
        pltpu.VMEM((CH * DIM,), jnp.float32),
        pltpu.VMEM((CH * DIM,), jnp.float32),
        pltpu.VMEM((CH,), jnp.float32),
        pltpu.VMEM((CH,), jnp.float32),
        pltpu.SemaphoreType.DMA,
        pltpu.SemaphoreType.DMA,
        pltpu.SemaphoreType.DMA,
        pltpu.SemaphoreType.DMA,
        pltpu.SemaphoreType.DMA,
        pltpu.SemaphoreType.DMA,
    ],
)
def _vegas_sc(y_hbm, tab_hbm, x_hbm, jac_hbm,
              tab_v, y0_v, y1_v, x0_v, x1_v, j0_v, j1_v,
              sy0, sy1, sx0, sx1, sj0, sj1):
    wid = lax.axis_index("s") * NC + lax.axis_index("c")
    base = wid * SPW

    pltpu.sync_copy(tab_hbm, tab_v)

    def start_y(c, y_v, sem):
        s0 = base + c * CH
        pltpu.async_copy(y_hbm.at[pl.ds(s0 * DIM, CH * DIM)], y_v, sem)

    def wait_y(y_v, sem):
        pltpu.make_async_copy(y_hbm.at[pl.ds(0, CH * DIM)], y_v, sem).wait()

    def start_out(c, x_v, jac_v, sem_x, sem_j):
        s0 = base + c * CH
        pltpu.async_copy(x_v, x_hbm.at[pl.ds(s0 * DIM, CH * DIM)], sem_x)
        pltpu.async_copy(jac_v, jac_hbm.at[pl.ds(s0, CH)], sem_j)

    def wait_out(x_v, jac_v, sem_x, sem_j):
        pltpu.make_async_copy(x_v, x_hbm.at[pl.ds(0, CH * DIM)], sem_x).wait()
        pltpu.make_async_copy(jac_v, jac_hbm.at[pl.ds(0, CH)], sem_j).wait()

    def compute_chunk(y_v, x_v, jac_v):
        def one_group(blk, sub):
            boff = blk * (BLK * DIM) + sub * L
            ys = [y_v[pl.ds(boff + d * BLK, L)] for d in range(DIM)]
            y1000s = [ys[d] * float(NINC) for d in range(DIM)]
            iycs = [y1000s[d].astype(jnp.int32) for d in range(DIM)]
            tws = [plsc.load_gather(tab_v.at[pl.ds(d * ROWP, ROWP)],
                                    [iycs[d]]) for d in range(DIM)]
            gvs = [lax.bitcast_convert_type(tws[d], jnp.float32)
                   for d in range(DIM)]
            ivs = [lax.bitcast_convert_type(tws[d] << 16,
                                            jnp.float32) for d in range(DIM)]
            xvs = [gvs[d] + ivs[d] * ys[d] for d in range(DIM)]
            for d in range(DIM):
                x_v[pl.ds(boff + d * BLK, L)] = xvs[d]
            p01 = ivs[0] * ivs[1]
            p23 = ivs[2] * ivs[3]
            p45 = ivs[4] * ivs[5]
            p67 = ivs[6] * ivs[7]
            jac = (p01 * p23) * (p45 * p67)
            jac_v[pl.ds(blk * BLK + sub * L, L)] = jac

        @plsc.parallel_loop(0, GROUPS, unroll=2)
        def _group_body(g):
            one_group(g >> 3, g & 7)

    NPAIR = NCHUNK // 2
    start_y(0, y0_v, sy0)

    def pair_body(p, carry):
        c0 = 2 * p
        wait_y(y0_v, sy0)
        start_y(c0 + 1, y1_v, sy1)

        @pl.when(p > 0)
        def _():
            wait_out(x0_v, j0_v, sx0, sj0)

        compute_chunk(y0_v, x0_v, j0_v)
        start_out(c0, x0_v, j0_v, sx0, sj0)

        wait_y(y1_v, sy1)

        @pl.when(p < NPAIR - 1)
        def _():
            start_y(c0 + 2, y0_v, sy0)

        @pl.when(p > 0)
        def _():
            wait_out(x1_v, j1_v, sx1, sj1)

        compute_chunk(y1_v, x1_v, j1_v)
        start_out(c0 + 1, x1_v, j1_v, sx1, sj1)
        return carry

    lax.fori_loop(0, NPAIR, pair_body, 0)
    wait_out(x0_v, j0_v, sx0, sj0)
    wait_out(x1_v, j1_v, sx1, sj1)


def kernel(y, grid, inc):
    yl = y.T.reshape(DIM, NBLK, BLK).transpose(1, 0, 2).reshape(-1)
    gridp = jnp.pad(grid, ((0, 0), (0, ROWP - NINC - 1)), mode="edge")
    incp = jnp.pad(inc, ((0, 0), (0, ROWP - NINC)), mode="edge")
    ii = jnp.arange(ROWP, dtype=jnp.float32)[None, :]
    a_val = gridp - ii * incp
    b_bits = lax.bitcast_convert_type(
        (incp * float(NINC)).astype(jnp.bfloat16), jnp.uint16).astype(jnp.uint32)
    t_bits = lax.bitcast_convert_type(a_val, jnp.uint32)
    h0 = (t_bits >> 16).astype(jnp.int32)
    cands = [jnp.clip(h0 + o, 0, 0xFFFF).astype(jnp.uint32)
             for o in (-1, 0, 1)]
    cvals = [lax.bitcast_convert_type((h << 16) | b_bits, jnp.float32)
             for h in cands]
    errs = [jnp.abs(v - a_val) for v in cvals]
    words = [(h << 16) | b_bits for h in cands]
    w01 = jnp.where(errs[0] < errs[1], words[0], words[1])
    e01 = jnp.minimum(errs[0], errs[1])
    best = jnp.where(errs[2] < e01, words[2], w01)
    tab = lax.bitcast_convert_type(best, jnp.int32).reshape(-1)
    xl, jac = _vegas_sc(yl, tab)
    x = xl.reshape(NBLK, DIM, BLK).transpose(1, 0, 2).reshape(DIM, BATCH).T
    return x, jac


# speedup vs baseline: 1.1349x; 1.0034x over previous
import functools

import jax
import jax.numpy as jnp
from jax import lax
from jax.experimental import pallas as pl
from jax.experimental.pallas import tpu as pltpu
from jax.experimental.pallas import tpu_sc as plsc

BATCH = 1048576
DIM = 8
NINC = 1000
ROWP = 1008
BLK = 128
NBLK = BATCH // BLK

NC = 2
NS = 16
NW = NC * NS
L = 16

SPW = BATCH // NW
CH = 2048
NCHUNK = SPW // CH
GROUPS = CH // L

_mesh = plsc.VectorSubcoreMesh(core_axis_name="c", subcore_axis_name="s")


@functools.partial(
    pl.kernel,
    mesh=_mesh,
    compiler_params=pltpu.CompilerParams(needs_layout_passes=False),
    out_type=[
        jax.ShapeDtypeStruct((BATCH * DIM,), jnp.float32),
        jax.ShapeDtypeStruct((BATCH,), jnp.float32),
    ],
    scratch_types=[
        pltpu.VMEM((DIM * ROWP,), jnp.int32),
        pltpu.VMEM((CH * DIM,), jnp.float32),
        pltpu.VMEM((CH * DIM,), jnp.float32),
        pltpu.VMEM((CH * DIM,), jnp.float32),
        pltpu.VMEM((CH * DIM,), jnp.float32),
        pltpu.VMEM((CH,), jnp.float32),
        pltpu.VMEM((CH,), jnp.float32),
        pltpu.SemaphoreType.DMA,
        pltpu.SemaphoreType.DMA,
        pltpu.SemaphoreType.DMA,
        pltpu.SemaphoreType.DMA,
        pltpu.SemaphoreType.DMA,
        pltpu.SemaphoreType.DMA,
    ],
)
def _vegas_sc(y_hbm, tab_hbm, x_hbm, jac_hbm,
              tab_v, y0_v, y1_v, x0_v, x1_v, j0_v, j1_v,
              sy0, sy1, sx0, sx1, sj0, sj1):
    wid = lax.axis_index("s") * NC + lax.axis_index("c")
    base = wid * SPW

    def start_y(c, y_v, sem):
        s0 = base + c * CH
        pltpu.async_copy(y_hbm.at[pl.ds(s0 * DIM, CH * DIM)], y_v, sem)

    def wait_y(y_v, sem):
        pltpu.make_async_copy(y_hbm.at[pl.ds(0, CH * DIM)], y_v, sem).wait()

    def start_out(c, x_v, jac_v, sem_x, sem_j):
        s0 = base + c * CH
        pltpu.async_copy(x_v, x_hbm.at[pl.ds(s0 * DIM, CH * DIM)], sem_x)
        pltpu.async_copy(jac_v, jac_hbm.at[pl.ds(s0, CH)], sem_j)

    def wait_out(x_v, jac_v, sem_x, sem_j):
        pltpu.make_async_copy(x_v, x_hbm.at[pl.ds(0, CH * DIM)], sem_x).wait()
        pltpu.make_async_copy(jac_v, jac_hbm.at[pl.ds(0, CH)], sem_j).wait()

    def compute_chunk(y_v, x_v, jac_v):
        def one_group(blk, sub):
            boff = blk * (BLK * DIM) + sub * L
            ys = [y_v[pl.ds(boff + d * BLK, L)] for d in range(DIM)]
            y1000s = [ys[d] * float(NINC) for d in range(DIM)]
            iycs = [y1000s[d].astype(jnp.int32) for d in range(DIM)]
            tws = [plsc.load_gather(tab_v.at[pl.ds(d * ROWP, ROWP)],
                                    [iycs[d]]) for d in range(DIM)]
            gvs = [lax.bitcast_convert_type(tws[d], jnp.float32)
                   for d in range(DIM)]
            ivs = [lax.bitcast_convert_type(tws[d] << 16,
                                            jnp.float32) for d in range(DIM)]
            xvs = [gvs[d] + ivs[d] * ys[d] for d in range(DIM)]
            for d in range(DIM):
                x_v[pl.ds(boff + d * BLK, L)] = xvs[d]
            p01 = ivs[0] * ivs[1]
            p23 = ivs[2] * ivs[3]
            p45 = ivs[4] * ivs[5]
            p67 = ivs[6] * ivs[7]
            jac = (p01 * p23) * (p45 * p67)
            jac_v[pl.ds(blk * BLK + sub * L, L)] = jac

        @plsc.parallel_loop(0, GROUPS, unroll=2)
        def _group_body(g):
            one_group(g >> 3, g & 7)

    NPAIR = NCHUNK // 2
    start_y(0, y0_v, sy0)
    pltpu.sync_copy(tab_hbm, tab_v)

    def pair_body(p, carry):
        c0 = 2 * p
        wait_y(y0_v, sy0)
        start_y(c0 + 1, y1_v, sy1)

        @pl.when(p > 0)
        def _():
            wait_out(x0_v, j0_v, sx0, sj0)

        compute_chunk(y0_v, x0_v, j0_v)
        start_out(c0, x0_v, j0_v, sx0, sj0)

        wait_y(y1_v, sy1)

        @pl.when(p < NPAIR - 1)
        def _():
            start_y(c0 + 2, y0_v, sy0)

        @pl.when(p > 0)
        def _():
            wait_out(x1_v, j1_v, sx1, sj1)

        compute_chunk(y1_v, x1_v, j1_v)
        start_out(c0 + 1, x1_v, j1_v, sx1, sj1)
        return carry

    lax.fori_loop(0, NPAIR, pair_body, 0)
    wait_out(x0_v, j0_v, sx0, sj0)
    wait_out(x1_v, j1_v, sx1, sj1)


def kernel(y, grid, inc):
    yl = y.T.reshape(DIM, NBLK, BLK).transpose(1, 0, 2).reshape(-1)
    gridp = jnp.pad(grid, ((0, 0), (0, ROWP - NINC - 1)), mode="edge")
    incp = jnp.pad(inc, ((0, 0), (0, ROWP - NINC)), mode="edge")
    ii = jnp.arange(ROWP, dtype=jnp.float32)[None, :]
    a_val = gridp - ii * incp
    b_bits = lax.bitcast_convert_type(
        (incp * float(NINC)).astype(jnp.bfloat16), jnp.uint16).astype(jnp.uint32)
    t_bits = lax.bitcast_convert_type(a_val, jnp.uint32)
    h0 = (t_bits >> 16).astype(jnp.int32)
    cands = [jnp.clip(h0 + o, 0, 0xFFFF).astype(jnp.uint32)
             for o in (-1, 0, 1)]
    cvals = [lax.bitcast_convert_type((h << 16) | b_bits, jnp.float32)
             for h in cands]
    errs = [jnp.abs(v - a_val) for v in cvals]
    words = [(h << 16) | b_bits for h in cands]
    w01 = jnp.where(errs[0] < errs[1], words[0], words[1])
    e01 = jnp.minimum(errs[0], errs[1])
    best = jnp.where(errs[2] < e01, words[2], w01)
    tab = lax.bitcast_convert_type(best, jnp.int32).reshape(-1)
    xl, jac = _vegas_sc(yl, tab)
    x = xl.reshape(NBLK, DIM, BLK).transpose(1, 0, 2).reshape(DIM, BATCH).T
    return x, jac
